# baseline (speedup 1.0000x reference)
import jax
import jax.numpy as jnp
from jax import lax
from jax.experimental import pallas as pl
from jax.experimental.pallas import tpu as pltpu

B = 16
T = 4
P = 1025
H = 1280
R = 9
K = 5
N = B * T


def _body(ids_ref, gate_ref, emb_ref, h_hbm, o_hbm, in_buf, out_buf, in_sems, out_sems):
    g = jnp.tanh(gate_ref[0])

    def in_copy(i, s):
        b = lax.div(i, T)
        t = lax.rem(i, T)
        return pltpu.make_async_copy(h_hbm.at[b, t], in_buf.at[s], in_sems.at[s])

    def out_copy(i, s):
        b = lax.div(i, T)
        t = lax.rem(i, T)
        return pltpu.make_async_copy(out_buf.at[s], o_hbm.at[b, t], out_sems.at[s])

    for j in range(K):
        in_copy(j, j).start()

    def step(i, carry):
        s = lax.rem(i, K)

        @pl.when(i >= K)
        def _():
            out_copy(i - K, s).wait()

        in_copy(i, s).wait()

        b = lax.div(i, T)
        t = lax.rem(i, T)
        row = ids_ref[b]
        e = emb_ref[row, t]
        out_buf[s] = in_buf[s] + e * g

        out_copy(i, s).start()

        @pl.when(i + K < N)
        def _():
            in_copy(i + K, s).start()

        return carry

    lax.fori_loop(0, N, step, 0)

    for j in range(K):
        i = N - K + j
        out_copy(i, i % K).wait()


def kernel(hidden_state, aspect_ratio_ids, embedding_weight, gate):
    ids = aspect_ratio_ids.astype(jnp.int32)
    emb = embedding_weight.reshape(R, T, 1, H)

    return pl.pallas_call(
        _body,
        in_specs=[
            pl.BlockSpec(memory_space=pltpu.SMEM),
            pl.BlockSpec(memory_space=pltpu.SMEM),
            pl.BlockSpec(memory_space=pltpu.VMEM),
            pl.BlockSpec(memory_space=pl.ANY),
        ],
        out_specs=pl.BlockSpec(memory_space=pl.ANY),
        out_shape=jax.ShapeDtypeStruct((B, T, P, H), jnp.float32),
        scratch_shapes=[
            pltpu.VMEM((K, P, H), jnp.float32),
            pltpu.VMEM((K, P, H), jnp.float32),
            pltpu.SemaphoreType.DMA((K,)),
            pltpu.SemaphoreType.DMA((K,)),
        ],
        compiler_params=pltpu.CompilerParams(
            vmem_limit_bytes=63 * 1024 * 1024,
        ),
    )(ids, gate, emb, hidden_state)


# speedup vs baseline: 4.1763x; 4.1763x over previous
import jax
import jax.numpy as jnp
from jax.experimental import pallas as pl
from jax.experimental.pallas import tpu as pltpu

B = 16
T = 4
P = 1025
H = 1280
R = 9
PB = 205


def _body(ids_ref, gate_ref, h_ref, emb_ref, o_ref):
    g = jnp.tanh(gate_ref[0])
    o_ref[...] = h_ref[...] + emb_ref[...] * g


def kernel(hidden_state, aspect_ratio_ids, embedding_weight, gate):
    ids = aspect_ratio_ids.astype(jnp.int32)
    hp = jnp.transpose(hidden_state, (0, 2, 1, 3))
    emb = embedding_weight.reshape(R, 1, T, H)

    grid_spec = pltpu.PrefetchScalarGridSpec(
        num_scalar_prefetch=2,
        grid=(B, P // PB),
        in_specs=[
            pl.BlockSpec((1, PB, T, H), lambda b, p, ids, gate: (b, p, 0, 0)),
            pl.BlockSpec((1, 1, T, H), lambda b, p, ids, gate: (ids[b], 0, 0, 0)),
        ],
        out_specs=pl.BlockSpec((1, PB, T, H), lambda b, p, ids, gate: (b, p, 0, 0)),
    )

    out = pl.pallas_call(
        _body,
        grid_spec=grid_spec,
        out_shape=jax.ShapeDtypeStruct((B, P, T, H), jnp.float32),
    )(ids, gate, hp, emb)
    return jnp.transpose(out, (0, 2, 1, 3))
